# Initial kernel scaffold; baseline (speedup 1.0000x reference)
import jax
import jax.numpy as jnp
from jax.experimental import pallas as pl


def kernel(indices, values, x):
    raise NotImplementedError("write your pallas kernel here")


# speedup vs baseline: 2.9917x; 2.9917x over previous
import functools

import jax
import jax.numpy as jnp
from jax import lax
from jax.experimental import pallas as pl
from jax.experimental.pallas import tpu as pltpu
from jax.experimental.pallas import tpu_sc as plsc

N = 10000
E = 320000
D = 128

NC = 2
NS = 16
DH = D // NC

EW = 100
CH = 8
ROWS = E // EW
ROWS_PER_TILE = ROWS // NS
N_CHUNKS = ROWS_PER_TILE // CH
GROWS = CH * EW

WCH = 200
N_WCH = N // WCH
WCH_PER_TILE = -(-N_WCH // NS)


def _sc_body(x2_hbm, row_hbm, col_hbm, val_hbm, out_hbm,
             row_v, col_v, val_v, g_v, acc_sh, sem):
  c = lax.axis_index("c")
  s = lax.axis_index("s")

  z = jnp.zeros((16,), jnp.float32)

  @pl.loop(0, WCH)
  def _zero(r):
    for q in range(DH // 16):
      g_v[r, pl.ds(q * 16, 16)] = z

  for t in range(WCH_PER_TILE):
    m = s + t * NS

    @pl.when(m < N_WCH)
    def _():
      pltpu.sync_copy(g_v.at[pl.ds(0, WCH)], acc_sh.at[pl.ds(m * WCH, WCH)])

  plsc.subcore_barrier()

  @pl.loop(0, N_CHUNKS)
  def _chunk(i):
    r0 = s * ROWS_PER_TILE + i * CH
    e0 = s * ROWS_PER_TILE * EW + i * GROWS
    pltpu.sync_copy(row_hbm.at[pl.ds(r0, CH)], row_v)
    pltpu.sync_copy(col_hbm.at[pl.ds(r0, CH)], col_v)
    pltpu.sync_copy(val_hbm.at[pl.ds(e0, GROWS)], val_v)

    for j in range(CH):
      pltpu.async_copy(x2_hbm.at[c].at[col_v.at[j]],
                       g_v.at[pl.ds(j * EW, EW)], sem).wait()

    @pl.loop(0, GROWS, step=16)
    def _scale(r16):
      vv = val_v[pl.ds(r16, 16)]
      for l in range(16):
        v = vv[l]
        for q in range(DH // 16):
          g_v[r16 + l, pl.ds(q * 16, 16)] = g_v[r16 + l, pl.ds(q * 16, 16)] * v

    for j in range(CH):
      pltpu.sync_copy(g_v.at[pl.ds(j * EW, EW)],
                      acc_sh.at[row_v.at[j]], add=True)

  plsc.subcore_barrier()

  for t in range(WCH_PER_TILE):
    m = s + t * NS

    @pl.when(m < N_WCH)
    def _():
      r = m * WCH
      pltpu.sync_copy(acc_sh.at[pl.ds(r, WCH)], g_v.at[pl.ds(0, WCH)])
      pltpu.sync_copy(g_v.at[pl.ds(0, WCH)], out_hbm.at[c].at[pl.ds(r, WCH)])


@jax.jit
def _run(x2, row2, col2, val1):
  mesh = plsc.VectorSubcoreMesh(core_axis_name="c", subcore_axis_name="s")
  f = pl.kernel(
      _sc_body,
      out_type=jax.ShapeDtypeStruct((NC, N, DH), jnp.float32),
      mesh=mesh,
      scratch_types=[
          pltpu.VMEM((CH, EW), jnp.int32),
          pltpu.VMEM((CH, EW), jnp.int32),
          pltpu.VMEM((GROWS,), jnp.float32),
          pltpu.VMEM((GROWS, DH), jnp.float32),
          pltpu.VMEM_SHARED((N, DH), jnp.float32),
          pltpu.SemaphoreType.DMA,
      ],
      compiler_params=pltpu.CompilerParams(use_tc_tiling_on_sc=False),
  )
  return f(x2, row2, col2, val1)


def kernel(indices, values, x):
  row = indices[0].astype(jnp.int32).reshape(ROWS, EW)
  col = indices[1].astype(jnp.int32).reshape(ROWS, EW)
  val = values.astype(jnp.float32).reshape(E)
  x2 = x.reshape(N, NC, DH).transpose(1, 0, 2)
  out2 = _run(x2, row, col, val)
  return out2.transpose(1, 0, 2).reshape(N, D)


# speedup vs baseline: 4.4248x; 1.4790x over previous
import functools

import jax
import jax.numpy as jnp
from jax import lax
from jax.experimental import pallas as pl
from jax.experimental.pallas import tpu as pltpu
from jax.experimental.pallas import tpu_sc as plsc

N = 10000
E = 320000
D = 128

NC = 2
NS = 16
DH = D // NC

EW = 100
CH = 4
ROWS = E // EW
ROWS_PER_TILE = ROWS // NS
E_PER_TILE = ROWS_PER_TILE * EW
N_CHUNKS = ROWS_PER_TILE // CH
GROWS = CH * EW

WCH = 200
N_WCH = N // WCH
WCH_PER_TILE = -(-N_WCH // NS)


def _sc_body(x2_hbm, row_hbm, col_hbm, val_hbm, out_hbm,
             col_v, row_v0, row_v1, val_v0, val_v1, g0, g1, acc_sh,
             gsem0, gsem1, ssem0, ssem1):
  c = lax.axis_index("c")
  s = lax.axis_index("s")

  gbufs = (g0, g1)
  rbufs = (row_v0, row_v1)
  vbufs = (val_v0, val_v1)
  gsems = (gsem0, gsem1)
  ssems = (ssem0, ssem1)

  pltpu.async_copy(col_hbm.at[pl.ds(s * ROWS_PER_TILE, ROWS_PER_TILE)],
                   col_v, gsem0)

  z = jnp.zeros((16,), jnp.float32)

  @pl.loop(0, WCH)
  def _zero(r):
    for q in range(DH // 16):
      g0[r, pl.ds(q * 16, 16)] = z

  for t in range(WCH_PER_TILE):
    m = s + t * NS

    @pl.when(m < N_WCH)
    def _():
      pltpu.sync_copy(g0.at[pl.ds(0, WCH)], acc_sh.at[pl.ds(m * WCH, WCH)])

  pltpu.make_async_copy(
      col_hbm.at[pl.ds(0, ROWS_PER_TILE)], col_v, gsem0).wait()

  def fire_chunk(i, b):
    pltpu.async_copy(row_hbm.at[pl.ds(s * ROWS_PER_TILE + i * CH, CH)],
                     rbufs[b], gsems[b])
    pltpu.async_copy(val_hbm.at[pl.ds(s * E_PER_TILE + i * GROWS, GROWS)],
                     vbufs[b], gsems[b])
    for j in range(CH):
      pltpu.async_copy(x2_hbm.at[c].at[col_v.at[i * CH + j]],
                       gbufs[b].at[pl.ds(j * EW, EW)], gsems[b])

  def drain_chunk(b):
    pltpu.make_async_copy(
        row_hbm.at[pl.ds(0, CH)], rbufs[b], gsems[b]).wait()
    pltpu.make_async_copy(
        val_hbm.at[pl.ds(0, GROWS)], vbufs[b], gsems[b]).wait()
    for j in range(CH):
      pltpu.make_async_copy(x2_hbm.at[c].at[col_v.at[j]],
                            gbufs[b].at[pl.ds(j * EW, EW)], gsems[b]).wait()

  def fire_scatters(b):
    for j in range(CH):
      pltpu.async_copy(gbufs[b].at[pl.ds(j * EW, EW)],
                       acc_sh.at[rbufs[b].at[j]], ssems[b], add=True)

  def drain_scatters(b):
    for j in range(CH):
      pltpu.make_async_copy(gbufs[b].at[pl.ds(j * EW, EW)],
                            acc_sh.at[rbufs[b].at[j]], ssems[b]).wait()

  def scale(b):
    g = gbufs[b]
    vb = vbufs[b]

    @pl.loop(0, GROWS, step=16)
    def _scale(r16):
      vv = vb[pl.ds(r16, 16)]
      for l in range(16):
        v = vv[l]
        for q in range(DH // 16):
          g[r16 + l, pl.ds(q * 16, 16)] = g[r16 + l, pl.ds(q * 16, 16)] * v

  def phase(i, b):
    nb = 1 - b
    @pl.when(i >= 1)
    def _():
      drain_scatters(nb)

    @pl.when(i + 1 < N_CHUNKS)
    def _():
      fire_chunk(i + 1, nb)

    drain_chunk(b)
    scale(b)
    fire_scatters(b)

  fire_chunk(0, 0)
  plsc.subcore_barrier()

  @pl.loop(0, N_CHUNKS, step=2)
  def _pair(i):
    phase(i, 0)
    phase(i + 1, 1)

  drain_scatters(1)
  plsc.subcore_barrier()

  for t in range(WCH_PER_TILE):
    m = s + t * NS

    @pl.when(m < N_WCH)
    def _():
      r = m * WCH
      pltpu.sync_copy(acc_sh.at[pl.ds(r, WCH)], g0.at[pl.ds(0, WCH)])
      pltpu.sync_copy(g0.at[pl.ds(0, WCH)], out_hbm.at[c].at[pl.ds(r, WCH)])


@jax.jit
def _run(x2, row2, col2, val1):
  mesh = plsc.VectorSubcoreMesh(core_axis_name="c", subcore_axis_name="s")
  f = pl.kernel(
      _sc_body,
      out_type=jax.ShapeDtypeStruct((NC, N, DH), jnp.float32),
      mesh=mesh,
      scratch_types=[
          pltpu.VMEM((ROWS_PER_TILE, EW), jnp.int32),
          pltpu.VMEM((CH, EW), jnp.int32),
          pltpu.VMEM((CH, EW), jnp.int32),
          pltpu.VMEM((GROWS,), jnp.float32),
          pltpu.VMEM((GROWS,), jnp.float32),
          pltpu.VMEM((GROWS, DH), jnp.float32),
          pltpu.VMEM((GROWS, DH), jnp.float32),
          pltpu.VMEM_SHARED((N, DH), jnp.float32),
          pltpu.SemaphoreType.DMA,
          pltpu.SemaphoreType.DMA,
          pltpu.SemaphoreType.DMA,
          pltpu.SemaphoreType.DMA,
      ],
      compiler_params=pltpu.CompilerParams(use_tc_tiling_on_sc=False),
  )
  return f(x2, row2, col2, val1)


def kernel(indices, values, x):
  row = indices[0].astype(jnp.int32).reshape(ROWS, EW)
  col = indices[1].astype(jnp.int32).reshape(ROWS, EW)
  val = values.astype(jnp.float32).reshape(E)
  x2 = x.reshape(N, NC, DH).transpose(1, 0, 2)
  out2 = _run(x2, row, col, val)
  return out2.transpose(1, 0, 2).reshape(N, D)


# speedup vs baseline: 4.6227x; 1.0447x over previous
import functools

import jax
import jax.numpy as jnp
from jax import lax
from jax.experimental import pallas as pl
from jax.experimental.pallas import tpu as pltpu
from jax.experimental.pallas import tpu_sc as plsc

N = 10000
E = 320000
D = 128

NC = 2
NS = 16
DH = D // NC

EW = 100
CH = 4
ROWS = E // EW
ROWS_PER_TILE = ROWS // NS
E_PER_TILE = ROWS_PER_TILE * EW
N_CHUNKS = ROWS_PER_TILE // CH
GROWS = CH * EW

WCH = 200
N_WCH = N // WCH
WCH_PER_TILE = -(-N_WCH // NS)


def _sc_body(x2_hbm, row_hbm, col_hbm, val_hbm, out_hbm,
             col_v, row_v0, row_v1, val_v0, val_v1, g0, g1, acc_sh,
             gsem0, gsem1, ssem0, ssem1):
  c = lax.axis_index("c")
  s = lax.axis_index("s")

  gbufs = (g0, g1)
  rbufs = (row_v0, row_v1)
  vbufs = (val_v0, val_v1)
  gsems = (gsem0, gsem1)
  ssems = (ssem0, ssem1)

  pltpu.async_copy(col_hbm.at[pl.ds(s * ROWS_PER_TILE, ROWS_PER_TILE)],
                   col_v, gsem0)

  z = jnp.zeros((16,), jnp.float32)

  @pl.loop(0, WCH)
  def _zero(r):
    for q in range(DH // 16):
      g0[r, pl.ds(q * 16, 16)] = z

  for t in range(WCH_PER_TILE):
    m = s + t * NS

    @pl.when(m < N_WCH)
    def _():
      pltpu.sync_copy(g0.at[pl.ds(0, WCH)], acc_sh.at[pl.ds(m * WCH, WCH)])

  pltpu.make_async_copy(
      col_hbm.at[pl.ds(0, ROWS_PER_TILE)], col_v, gsem0).wait()

  def fire_chunk(i, b):
    pltpu.async_copy(row_hbm.at[pl.ds(s * ROWS_PER_TILE + i * CH, CH)],
                     rbufs[b], gsems[b])
    pltpu.async_copy(val_hbm.at[pl.ds(s * E_PER_TILE + i * GROWS, GROWS)],
                     vbufs[b], gsems[b])
    for j in range(CH):
      pltpu.async_copy(x2_hbm.at[c].at[col_v.at[i * CH + j]],
                       gbufs[b].at[pl.ds(j * EW, EW)], gsems[b])

  def drain_chunk(b):
    pltpu.make_async_copy(
        row_hbm.at[pl.ds(0, CH)], rbufs[b], gsems[b]).wait()
    pltpu.make_async_copy(
        val_hbm.at[pl.ds(0, GROWS)], vbufs[b], gsems[b]).wait()
    for j in range(CH):
      pltpu.make_async_copy(x2_hbm.at[c].at[col_v.at[j]],
                            gbufs[b].at[pl.ds(j * EW, EW)], gsems[b]).wait()

  def fire_scatters(b):
    for j in range(CH):
      pltpu.async_copy(gbufs[b].at[pl.ds(j * EW, EW)],
                       acc_sh.at[rbufs[b].at[j]], ssems[b], add=True)

  def drain_scatters(b):
    for j in range(CH):
      pltpu.make_async_copy(gbufs[b].at[pl.ds(j * EW, EW)],
                            acc_sh.at[rbufs[b].at[j]], ssems[b]).wait()

  def scale(b):
    g = gbufs[b]
    vb = vbufs[b]

    @pl.loop(0, GROWS, step=16)
    def _scale(r16):
      vv = vb[pl.ds(r16, 16)]
      for l in range(16):
        v = vv[l]
        for q in range(DH // 16):
          g[r16 + l, pl.ds(q * 16, 16)] = g[r16 + l, pl.ds(q * 16, 16)] * v

  def phase(i, b):
    nb = 1 - b
    @pl.when(i >= 1)
    def _():
      drain_scatters(nb)

    @pl.when(i + 1 < N_CHUNKS)
    def _():
      fire_chunk(i + 1, nb)

    drain_chunk(b)
    scale(b)
    fire_scatters(b)

  fire_chunk(0, 0)
  plsc.subcore_barrier()

  @pl.loop(0, N_CHUNKS, step=2)
  def _pair(i):
    phase(i, 0)
    phase(i + 1, 1)

  drain_scatters(1)
  plsc.subcore_barrier()

  for t in range(WCH_PER_TILE):
    m = s + t * NS

    @pl.when(m < N_WCH)
    def _():
      r = m * WCH
      pltpu.sync_copy(acc_sh.at[pl.ds(r, WCH)], g0.at[pl.ds(0, WCH)])
      pltpu.sync_copy(g0.at[pl.ds(0, WCH)],
                      out_hbm.at[pl.ds(r, WCH), pl.ds(c * DH, DH)])


@jax.jit
def _run(x2, row2, col2, val1):
  mesh = plsc.VectorSubcoreMesh(core_axis_name="c", subcore_axis_name="s")
  f = pl.kernel(
      _sc_body,
      out_type=jax.ShapeDtypeStruct((N, D), jnp.float32),
      mesh=mesh,
      scratch_types=[
          pltpu.VMEM((ROWS_PER_TILE, EW), jnp.int32),
          pltpu.VMEM((CH, EW), jnp.int32),
          pltpu.VMEM((CH, EW), jnp.int32),
          pltpu.VMEM((GROWS,), jnp.float32),
          pltpu.VMEM((GROWS,), jnp.float32),
          pltpu.VMEM((GROWS, DH), jnp.float32),
          pltpu.VMEM((GROWS, DH), jnp.float32),
          pltpu.VMEM_SHARED((N, DH), jnp.float32),
          pltpu.SemaphoreType.DMA,
          pltpu.SemaphoreType.DMA,
          pltpu.SemaphoreType.DMA,
          pltpu.SemaphoreType.DMA,
      ],
      compiler_params=pltpu.CompilerParams(use_tc_tiling_on_sc=False),
  )
  return f(x2, row2, col2, val1)


def kernel(indices, values, x):
  row = indices[0].astype(jnp.int32).reshape(ROWS, EW)
  col = indices[1].astype(jnp.int32).reshape(ROWS, EW)
  val = values.astype(jnp.float32).reshape(E)
  x2 = x.reshape(N, NC, DH).transpose(1, 0, 2)
  return _run(x2, row, col, val)


# speedup vs baseline: 4.8105x; 1.0406x over previous
import functools

import jax
import jax.numpy as jnp
from jax import lax
from jax.experimental import pallas as pl
from jax.experimental.pallas import tpu as pltpu
from jax.experimental.pallas import tpu_sc as plsc

N = 10000
E = 320000
D = 128

NC = 2
NS = 16
DH = D // NC

EW = 100
CH = 4
GW = 200
NG = 2
ROWS = E // EW
ROWS_PER_TILE = ROWS // NS
E_PER_TILE = ROWS_PER_TILE * EW
N_CHUNKS = ROWS_PER_TILE // CH
GROWS = CH * EW

WCH = 200
N_WCH = N // WCH
WCH_PER_TILE = -(-N_WCH // NS)


def _sc_body(x2_hbm, row_hbm, col_hbm, val_hbm, out_hbm,
             col_v, row_v0, row_v1, val_v0, val_v1, g0, g1, acc_sh,
             gsem0, gsem1, ssem0, ssem1):
  c = lax.axis_index("c")
  s = lax.axis_index("s")

  gbufs = (g0, g1)
  rbufs = (row_v0, row_v1)
  vbufs = (val_v0, val_v1)
  gsems = (gsem0, gsem1)
  ssems = (ssem0, ssem1)

  pltpu.async_copy(col_hbm.at[pl.ds(s * E_PER_TILE, E_PER_TILE)],
                   col_v, gsem0)

  z = jnp.zeros((16,), jnp.float32)

  @pl.loop(0, WCH)
  def _zero(r):
    for q in range(DH // 16):
      g0[r, pl.ds(q * 16, 16)] = z

  for t in range(WCH_PER_TILE):
    m = s + t * NS

    @pl.when(m < N_WCH)
    def _():
      pltpu.sync_copy(g0.at[pl.ds(0, WCH)], acc_sh.at[pl.ds(m * WCH, WCH)])

  pltpu.make_async_copy(
      col_hbm.at[pl.ds(0, E_PER_TILE)], col_v, gsem0).wait()

  @pl.loop(0, E_PER_TILE, step=16)
  def _xform(k):
    col_v[pl.ds(k, 16)] = col_v[pl.ds(k, 16)] * 2 + c

  def fire_chunk(i, b):
    pltpu.async_copy(row_hbm.at[pl.ds(s * ROWS_PER_TILE + i * CH, CH)],
                     rbufs[b], gsems[b])
    pltpu.async_copy(val_hbm.at[pl.ds(s * E_PER_TILE + i * GROWS, GROWS)],
                     vbufs[b], gsems[b])
    for j in range(NG):
      pltpu.async_copy(
          x2_hbm.at[col_v.at[pl.ds(i * GROWS + j * GW, GW)]],
          gbufs[b].at[pl.ds(j * GW, GW)], gsems[b])

  def drain_chunk(b):
    pltpu.make_async_copy(
        row_hbm.at[pl.ds(0, CH)], rbufs[b], gsems[b]).wait()
    pltpu.make_async_copy(
        val_hbm.at[pl.ds(0, GROWS)], vbufs[b], gsems[b]).wait()
    for j in range(NG):
      pltpu.make_async_copy(
          x2_hbm.at[col_v.at[pl.ds(j * GW, GW)]],
          gbufs[b].at[pl.ds(j * GW, GW)], gsems[b]).wait()

  def fire_scatters(b):
    for j in range(CH):
      pltpu.async_copy(gbufs[b].at[pl.ds(j * EW, EW)],
                       acc_sh.at[rbufs[b].at[j]], ssems[b], add=True)

  def drain_scatters(b):
    for j in range(CH):
      pltpu.make_async_copy(gbufs[b].at[pl.ds(j * EW, EW)],
                            acc_sh.at[rbufs[b].at[j]], ssems[b]).wait()

  def scale(b):
    g = gbufs[b]
    vb = vbufs[b]

    @pl.loop(0, GROWS, step=16)
    def _scale(r16):
      vv = vb[pl.ds(r16, 16)]
      for l in range(16):
        v = vv[l]
        for q in range(DH // 16):
          g[r16 + l, pl.ds(q * 16, 16)] = g[r16 + l, pl.ds(q * 16, 16)] * v

  def phase(i, b):
    nb = 1 - b
    @pl.when(i >= 1)
    def _():
      drain_scatters(nb)

    @pl.when(i + 1 < N_CHUNKS)
    def _():
      fire_chunk(i + 1, nb)

    drain_chunk(b)
    scale(b)
    fire_scatters(b)

  fire_chunk(0, 0)
  plsc.subcore_barrier()

  @pl.loop(0, N_CHUNKS, step=2)
  def _pair(i):
    phase(i, 0)
    phase(i + 1, 1)

  drain_scatters(1)
  plsc.subcore_barrier()

  for t in range(WCH_PER_TILE):
    m = s + t * NS

    @pl.when(m < N_WCH)
    def _():
      r = m * WCH
      pltpu.sync_copy(acc_sh.at[pl.ds(r, WCH)], g0.at[pl.ds(0, WCH)])
      pltpu.sync_copy(g0.at[pl.ds(0, WCH)],
                      out_hbm.at[pl.ds(r, WCH), pl.ds(c * DH, DH)])


@jax.jit
def _run(x2, row2, col2, val1):
  mesh = plsc.VectorSubcoreMesh(core_axis_name="c", subcore_axis_name="s")
  f = pl.kernel(
      _sc_body,
      out_type=jax.ShapeDtypeStruct((N, D), jnp.float32),
      mesh=mesh,
      scratch_types=[
          pltpu.VMEM((E_PER_TILE,), jnp.int32),
          pltpu.VMEM((CH, EW), jnp.int32),
          pltpu.VMEM((CH, EW), jnp.int32),
          pltpu.VMEM((GROWS,), jnp.float32),
          pltpu.VMEM((GROWS,), jnp.float32),
          pltpu.VMEM((GROWS, DH), jnp.float32),
          pltpu.VMEM((GROWS, DH), jnp.float32),
          pltpu.VMEM_SHARED((N, DH), jnp.float32),
          pltpu.SemaphoreType.DMA,
          pltpu.SemaphoreType.DMA,
          pltpu.SemaphoreType.DMA,
          pltpu.SemaphoreType.DMA,
      ],
      compiler_params=pltpu.CompilerParams(use_tc_tiling_on_sc=False),
  )
  return f(x2, row2, col2, val1)


def kernel(indices, values, x):
  row = indices[0].astype(jnp.int32).reshape(ROWS, EW)
  col = indices[1].astype(jnp.int32).reshape(E)
  val = values.astype(jnp.float32).reshape(E)
  x2 = x.reshape(N * NC, DH)
  return _run(x2, row, col, val)


# speedup vs baseline: 12.3577x; 2.5689x over previous
import functools

import jax
import jax.numpy as jnp
from jax import lax
from jax.experimental import pallas as pl
from jax.experimental.pallas import tpu as pltpu
from jax.experimental.pallas import tpu_sc as plsc

N = 10000
E = 320000
D = 128

NC = 2
NS = 16
DH = D // NC

EW = 100
CH = 4
GW = 200
NG = 2
ROWS = E // EW
ROWS_PER_TILE = ROWS // NS
E_PER_TILE = ROWS_PER_TILE * EW
N_CHUNKS = ROWS_PER_TILE // CH
GROWS = CH * EW

WCH = 200
N_WCH = N // WCH
WCH_PER_TILE = -(-N_WCH // NS)


def _sc_body(x2_hbm, row_hbm, col_hbm, val_hbm, out_hbm,
             col_v, row_v0, row_v1, val_v0, val_v1, g0, g1, acc_sh,
             gsem0, gsem1, ssem0, ssem1):
  c = lax.axis_index("c")
  s = lax.axis_index("s")

  gbufs = (g0, g1)
  rbufs = (row_v0, row_v1)
  vbufs = (val_v0, val_v1)
  gsems = (gsem0, gsem1)
  ssems = (ssem0, ssem1)

  pltpu.async_copy(col_hbm.at[pl.ds(s * E_PER_TILE, E_PER_TILE)],
                   col_v, gsem0)

  z = jnp.zeros((16,), jnp.float32)

  @pl.loop(0, WCH)
  def _zero(r):
    for q in range(DH // 16):
      g0[r, pl.ds(q * 16, 16)] = z

  for t in range(WCH_PER_TILE):
    m = s + t * NS

    @pl.when(m < N_WCH)
    def _():
      pltpu.sync_copy(g0.at[pl.ds(0, WCH)], acc_sh.at[pl.ds(m * WCH, WCH)])

  pltpu.make_async_copy(
      col_hbm.at[pl.ds(0, E_PER_TILE)], col_v, gsem0).wait()

  @pl.loop(0, E_PER_TILE, step=16)
  def _xform(k):
    col_v[pl.ds(k, 16)] = col_v[pl.ds(k, 16)] * 2 + c

  def fire_chunk(i, b):
    pltpu.async_copy(row_hbm.at[pl.ds(s * ROWS_PER_TILE + i * CH, CH)],
                     rbufs[b], gsems[b])
    pltpu.async_copy(val_hbm.at[pl.ds(s * E_PER_TILE + i * GROWS, GROWS)],
                     vbufs[b], gsems[b])
    for j in range(NG):
      pltpu.async_copy(
          x2_hbm.at[col_v.at[pl.ds(i * GROWS + j * GW, GW)]],
          gbufs[b].at[pl.ds(j * GW, GW)], gsems[b])

  def drain_chunk(b):
    pltpu.make_async_copy(
        row_hbm.at[pl.ds(0, CH)], rbufs[b], gsems[b]).wait()
    pltpu.make_async_copy(
        val_hbm.at[pl.ds(0, GROWS)], vbufs[b], gsems[b]).wait()
    for j in range(NG):
      pltpu.make_async_copy(
          x2_hbm.at[col_v.at[pl.ds(j * GW, GW)]],
          gbufs[b].at[pl.ds(j * GW, GW)], gsems[b]).wait()

  def fire_scatters(b):
    for j in range(CH):
      pltpu.async_copy(gbufs[b].at[pl.ds(j * EW, EW)],
                       acc_sh.at[rbufs[b].at[j]], ssems[b], add=True)

  def drain_scatters(b):
    for j in range(CH):
      pltpu.make_async_copy(gbufs[b].at[pl.ds(j * EW, EW)],
                            acc_sh.at[rbufs[b].at[j]], ssems[b]).wait()

  def scale(b):
    g = gbufs[b]
    vb = vbufs[b]

    @plsc.parallel_loop(0, GROWS, 16, unroll=2)
    def _scale(r16):
      vv = vb[pl.ds(r16, 16)]
      for l in range(16):
        v = vv[l]
        for q in range(DH // 16):
          g[r16 + l, pl.ds(q * 16, 16)] = g[r16 + l, pl.ds(q * 16, 16)] * v

  def phase(i, b):
    nb = 1 - b
    @pl.when(i >= 1)
    def _():
      drain_scatters(nb)

    @pl.when(i + 1 < N_CHUNKS)
    def _():
      fire_chunk(i + 1, nb)

    drain_chunk(b)
    scale(b)
    fire_scatters(b)

  fire_chunk(0, 0)
  plsc.subcore_barrier()

  @pl.loop(0, N_CHUNKS, step=2)
  def _pair(i):
    phase(i, 0)
    phase(i + 1, 1)

  drain_scatters(1)
  plsc.subcore_barrier()

  for t in range(WCH_PER_TILE):
    m = s + t * NS

    @pl.when(m < N_WCH)
    def _():
      r = m * WCH
      pltpu.sync_copy(acc_sh.at[pl.ds(r, WCH)], g0.at[pl.ds(0, WCH)])
      pltpu.sync_copy(g0.at[pl.ds(0, WCH)],
                      out_hbm.at[pl.ds(r, WCH), pl.ds(c * DH, DH)])


@jax.jit
def _run(x2, row2, col2, val1):
  mesh = plsc.VectorSubcoreMesh(core_axis_name="c", subcore_axis_name="s")
  f = pl.kernel(
      _sc_body,
      out_type=jax.ShapeDtypeStruct((N, D), jnp.float32),
      mesh=mesh,
      scratch_types=[
          pltpu.VMEM((E_PER_TILE,), jnp.int32),
          pltpu.VMEM((CH, EW), jnp.int32),
          pltpu.VMEM((CH, EW), jnp.int32),
          pltpu.VMEM((GROWS,), jnp.float32),
          pltpu.VMEM((GROWS,), jnp.float32),
          pltpu.VMEM((GROWS, DH), jnp.float32),
          pltpu.VMEM((GROWS, DH), jnp.float32),
          pltpu.VMEM_SHARED((N, DH), jnp.float32),
          pltpu.SemaphoreType.DMA,
          pltpu.SemaphoreType.DMA,
          pltpu.SemaphoreType.DMA,
          pltpu.SemaphoreType.DMA,
      ],
      compiler_params=pltpu.CompilerParams(use_tc_tiling_on_sc=False),
  )
  return f(x2, row2, col2, val1)


def kernel(indices, values, x):
  row = indices[0].astype(jnp.int32).reshape(ROWS, EW)
  col = indices[1].astype(jnp.int32).reshape(E)
  val = values.astype(jnp.float32).reshape(E)
  x2 = x.reshape(N * NC, DH)
  return _run(x2, row, col, val)


# speedup vs baseline: 13.4476x; 1.0882x over previous
import functools

import jax
import jax.numpy as jnp
from jax import lax
from jax.experimental import pallas as pl
from jax.experimental.pallas import tpu as pltpu
from jax.experimental.pallas import tpu_sc as plsc

N = 10000
E = 320000
D = 128

NC = 2
NS = 16
DH = D // NC

EW = 100
CH = 2
ROWS = E // EW
ROWS_PER_TILE = ROWS // NS
E_PER_TILE = ROWS_PER_TILE * EW
N_CHUNKS = ROWS_PER_TILE // CH
GROWS = CH * EW
GPAD = 208
NB = 4

WCH = 200
N_WCH = N // WCH
WCH_PER_TILE = -(-N_WCH // NS)


def _sc_body(x2_hbm, row_hbm, col_hbm, val_hbm, out_hbm,
             col_v, row_v0, row_v1, row_v2, row_v3,
             val_v0, val_v1, val_v2, val_v3, g0, g1, g2, g3, acc_sh,
             gsem0, gsem1, gsem2, gsem3, ssem0, ssem1, ssem2, ssem3):
  c = lax.axis_index("c")
  s = lax.axis_index("s")

  gbufs = (g0, g1, g2, g3)
  rbufs = (row_v0, row_v1, row_v2, row_v3)
  vbufs = (val_v0, val_v1, val_v2, val_v3)
  gsems = (gsem0, gsem1, gsem2, gsem3)
  ssems = (ssem0, ssem1, ssem2, ssem3)

  pltpu.async_copy(col_hbm.at[pl.ds(s * E_PER_TILE, E_PER_TILE)],
                   col_v, gsem0)

  z = jnp.zeros((16,), jnp.float32)

  @plsc.parallel_loop(0, WCH, 1)
  def _zero(r):
    for q in range(DH // 16):
      g0[r, pl.ds(q * 16, 16)] = z

  for t in range(WCH_PER_TILE):
    m = s + t * NS

    @pl.when(m < N_WCH)
    def _():
      pltpu.sync_copy(g0.at[pl.ds(0, WCH)], acc_sh.at[pl.ds(m * WCH, WCH)])

  pltpu.make_async_copy(
      col_hbm.at[pl.ds(0, E_PER_TILE)], col_v, gsem0).wait()

  @plsc.parallel_loop(0, E_PER_TILE, 16, unroll=4)
  def _xform(k):
    col_v[pl.ds(k, 16)] = col_v[pl.ds(k, 16)] * 2 + c

  def fire_chunk(i, b):
    pltpu.async_copy(row_hbm.at[pl.ds(s * ROWS_PER_TILE + i * CH, CH)],
                     rbufs[b], gsems[b])
    pltpu.async_copy(val_hbm.at[pl.ds(s * E_PER_TILE + i * GROWS, GROWS)],
                     vbufs[b].at[pl.ds(0, GROWS)], gsems[b])
    pltpu.async_copy(x2_hbm.at[col_v.at[pl.ds(i * GROWS, GROWS)]],
                     gbufs[b].at[pl.ds(0, GROWS)], gsems[b])

  def drain_chunk(b):
    pltpu.make_async_copy(
        row_hbm.at[pl.ds(0, CH)], rbufs[b], gsems[b]).wait()
    pltpu.make_async_copy(
        val_hbm.at[pl.ds(0, GROWS)], vbufs[b].at[pl.ds(0, GROWS)],
        gsems[b]).wait()
    pltpu.make_async_copy(
        x2_hbm.at[col_v.at[pl.ds(0, GROWS)]],
        gbufs[b].at[pl.ds(0, GROWS)], gsems[b]).wait()

  def fire_scatters(b):
    for j in range(CH):
      pltpu.async_copy(gbufs[b].at[pl.ds(j * EW, EW)],
                       acc_sh.at[rbufs[b].at[j]], ssems[b], add=True)

  def drain_scatters(b):
    for j in range(CH):
      pltpu.make_async_copy(gbufs[b].at[pl.ds(j * EW, EW)],
                            acc_sh.at[rbufs[b].at[j]], ssems[b]).wait()

  def scale(b):
    g = gbufs[b]
    vb = vbufs[b]

    @plsc.parallel_loop(0, GROWS, 16, unroll=2)
    def _scale(r16):
      vv = vb[pl.ds(r16, 16)]
      for l in range(16):
        v = vv[l]
        for q in range(DH // 16):
          g[r16 + l, pl.ds(q * 16, 16)] = g[r16 + l, pl.ds(q * 16, 16)] * v

  def phase(i, k):
    @pl.when(i >= 2)
    def _():
      drain_scatters((k + 2) % NB)

    @pl.when(i + 2 < N_CHUNKS)
    def _():
      fire_chunk(i + 2, (k + 2) % NB)

    drain_chunk(k)
    scale(k)
    fire_scatters(k)

  fire_chunk(0, 0)
  fire_chunk(1, 1)
  plsc.subcore_barrier()

  @pl.loop(0, N_CHUNKS, step=NB)
  def _quad(i):
    for k in range(NB):
      phase(i + k, k)

  drain_scatters((N_CHUNKS - 2) % NB)
  drain_scatters((N_CHUNKS - 1) % NB)
  plsc.subcore_barrier()

  for t in range(WCH_PER_TILE):
    m = s + t * NS

    @pl.when(m < N_WCH)
    def _():
      r = m * WCH
      pltpu.sync_copy(acc_sh.at[pl.ds(r, WCH)], g0.at[pl.ds(0, WCH)])
      pltpu.sync_copy(g0.at[pl.ds(0, WCH)],
                      out_hbm.at[pl.ds(r, WCH), pl.ds(c * DH, DH)])


@jax.jit
def _run(x2, row2, col1, val1):
  mesh = plsc.VectorSubcoreMesh(core_axis_name="c", subcore_axis_name="s")
  f = pl.kernel(
      _sc_body,
      out_type=jax.ShapeDtypeStruct((N, D), jnp.float32),
      mesh=mesh,
      scratch_types=(
          [pltpu.VMEM((E_PER_TILE,), jnp.int32)]
          + [pltpu.VMEM((CH, EW), jnp.int32)] * NB
          + [pltpu.VMEM((GPAD,), jnp.float32)] * NB
          + [pltpu.VMEM((GPAD, DH), jnp.float32)] * NB
          + [pltpu.VMEM_SHARED((N, DH), jnp.float32)]
          + [pltpu.SemaphoreType.DMA] * (2 * NB)
      ),
      compiler_params=pltpu.CompilerParams(use_tc_tiling_on_sc=False),
  )
  return f(x2, row2, col1, val1)


def kernel(indices, values, x):
  row = indices[0].astype(jnp.int32).reshape(ROWS, EW)
  col = indices[1].astype(jnp.int32).reshape(E)
  val = values.astype(jnp.float32).reshape(E)
  x2 = x.reshape(N * NC, DH)
  return _run(x2, row, col, val)


# speedup vs baseline: 13.4548x; 1.0005x over previous
import functools

import jax
import jax.numpy as jnp
from jax import lax
from jax.experimental import pallas as pl
from jax.experimental.pallas import tpu as pltpu
from jax.experimental.pallas import tpu_sc as plsc

N = 10000
E = 320000
D = 128

NC = 2
NS = 16
DH = D // NC

EW = 100
CH = 2
ROWS = E // EW
ROWS_PER_TILE = ROWS // NS
E_PER_TILE = ROWS_PER_TILE * EW
N_CHUNKS = ROWS_PER_TILE // CH
GROWS = CH * EW
GPAD = 208
NB = 4

WCH = 200
N_WCH = N // WCH
WCH_PER_TILE = -(-N_WCH // NS)


def _sc_body(x2_hbm, row_hbm, col_hbm, val_hbm, out_hbm,
             col_v, row_v0, row_v1, row_v2, row_v3,
             val_v0, val_v1, val_v2, val_v3, g0, g1, g2, g3, acc_sh,
             gsem0, gsem1, gsem2, gsem3, ssem0, ssem1, ssem2, ssem3):
  c = lax.axis_index("c")
  s = lax.axis_index("s")

  gbufs = (g0, g1, g2, g3)
  rbufs = (row_v0, row_v1, row_v2, row_v3)
  vbufs = (val_v0, val_v1, val_v2, val_v3)
  gsems = (gsem0, gsem1, gsem2, gsem3)
  ssems = (ssem0, ssem1, ssem2, ssem3)

  pltpu.async_copy(col_hbm.at[pl.ds(s * E_PER_TILE, E_PER_TILE)],
                   col_v, gsem0)

  z = jnp.zeros((16,), jnp.float32)

  @plsc.parallel_loop(0, WCH, 1)
  def _zero(r):
    for q in range(DH // 16):
      g0[r, pl.ds(q * 16, 16)] = z

  for t in range(WCH_PER_TILE):
    m = s + t * NS

    @pl.when(m < N_WCH)
    def _():
      pltpu.sync_copy(g0.at[pl.ds(0, WCH)], acc_sh.at[pl.ds(m * WCH, WCH)])

  pltpu.make_async_copy(
      col_hbm.at[pl.ds(0, E_PER_TILE)], col_v, gsem0).wait()

  @plsc.parallel_loop(0, E_PER_TILE, 16, unroll=4)
  def _xform(k):
    col_v[pl.ds(k, 16)] = col_v[pl.ds(k, 16)] * 2 + c

  def fire_chunk(i, b):
    pltpu.async_copy(row_hbm.at[pl.ds(s * ROWS_PER_TILE + i * CH, CH)],
                     rbufs[b], gsems[b])
    pltpu.async_copy(val_hbm.at[pl.ds(s * E_PER_TILE + i * GROWS, GROWS)],
                     vbufs[b].at[pl.ds(0, GROWS)], gsems[b])
    pltpu.async_copy(x2_hbm.at[col_v.at[pl.ds(i * GROWS, GROWS)]],
                     gbufs[b].at[pl.ds(0, GROWS)], gsems[b])

  def drain_chunk(b):
    pltpu.make_async_copy(
        row_hbm.at[pl.ds(0, CH)], rbufs[b], gsems[b]).wait()
    pltpu.make_async_copy(
        val_hbm.at[pl.ds(0, GROWS)], vbufs[b].at[pl.ds(0, GROWS)],
        gsems[b]).wait()
    pltpu.make_async_copy(
        x2_hbm.at[col_v.at[pl.ds(0, GROWS)]],
        gbufs[b].at[pl.ds(0, GROWS)], gsems[b]).wait()

  def fire_scatters(b):
    for j in range(CH):
      pltpu.async_copy(gbufs[b].at[pl.ds(j * EW, EW)],
                       acc_sh.at[rbufs[b].at[j]], ssems[b], add=True)

  def drain_scatters(b):
    for j in range(CH):
      pltpu.make_async_copy(gbufs[b].at[pl.ds(j * EW, EW)],
                            acc_sh.at[rbufs[b].at[j]], ssems[b]).wait()

  def scale(b):
    g = gbufs[b]
    vb = vbufs[b]

    @plsc.parallel_loop(0, GROWS, 16, unroll=2)
    def _scale(r16):
      vv = vb[pl.ds(r16, 16)]
      for l in range(16):
        v = vv[l]
        for q in range(DH // 16):
          g[r16 + l, pl.ds(q * 16, 16)] = g[r16 + l, pl.ds(q * 16, 16)] * v

  def phase(i, k):
    @pl.when(i >= 2)
    def _():
      drain_scatters((k + 2) % NB)

    @pl.when(i + 2 < N_CHUNKS)
    def _():
      fire_chunk(i + 2, (k + 2) % NB)

    drain_chunk(k)
    scale(k)
    fire_scatters(k)

  fire_chunk(0, 0)
  fire_chunk(1, 1)
  plsc.subcore_barrier()

  @pl.loop(0, N_CHUNKS, step=NB)
  def _quad(i):
    for k in range(NB):
      phase(i + k, k)

  drain_scatters((N_CHUNKS - 2) % NB)
  drain_scatters((N_CHUNKS - 1) % NB)
  plsc.subcore_barrier()

  for t in range(WCH_PER_TILE):
    m = s + t * NS

    @pl.when(m < N_WCH)
    def _():
      r = m * WCH
      pltpu.sync_copy(acc_sh.at[pl.ds(r, WCH)],
                      out_hbm.at[pl.ds(r, WCH), pl.ds(c * DH, DH)])


@jax.jit
def _run(x2, row2, col1, val1):
  mesh = plsc.VectorSubcoreMesh(core_axis_name="c", subcore_axis_name="s")
  f = pl.kernel(
      _sc_body,
      out_type=jax.ShapeDtypeStruct((N, D), jnp.float32),
      mesh=mesh,
      scratch_types=(
          [pltpu.VMEM((E_PER_TILE,), jnp.int32)]
          + [pltpu.VMEM((CH, EW), jnp.int32)] * NB
          + [pltpu.VMEM((GPAD,), jnp.float32)] * NB
          + [pltpu.VMEM((GPAD, DH), jnp.float32)] * NB
          + [pltpu.VMEM_SHARED((N, DH), jnp.float32)]
          + [pltpu.SemaphoreType.DMA] * (2 * NB)
      ),
      compiler_params=pltpu.CompilerParams(use_tc_tiling_on_sc=False),
  )
  return f(x2, row2, col1, val1)


def kernel(indices, values, x):
  row = indices[0].astype(jnp.int32).reshape(ROWS, EW)
  col = indices[1].astype(jnp.int32).reshape(E)
  val = values.astype(jnp.float32).reshape(E)
  x2 = x.reshape(N * NC, DH)
  return _run(x2, row, col, val)


# speedup vs baseline: 13.6174x; 1.0121x over previous
import functools

import jax
import jax.numpy as jnp
from jax import lax
from jax.experimental import pallas as pl
from jax.experimental.pallas import tpu as pltpu
from jax.experimental.pallas import tpu_sc as plsc

N = 10000
E = 320000
D = 128

NC = 2
NS = 16
DH = D // NC

EW = 100
CH = 2
ROWS = E // EW
ROWS_PER_TILE = ROWS // NS
E_PER_TILE = ROWS_PER_TILE * EW
N_CHUNKS = ROWS_PER_TILE // CH
GROWS = CH * EW
GPAD = 208
NB = 4

WCH = 200
N_WCH = N // WCH
WCH_PER_TILE = -(-N_WCH // NS)


def _sc_body(x2_hbm, row_hbm, col_hbm, val_hbm, out_hbm,
             col_v, row_v0, row_v1, row_v2, row_v3,
             val_v0, val_v1, val_v2, val_v3, g0, g1, g2, g3, acc_sh,
             gsem0, gsem1, gsem2, gsem3, ssem0, ssem1, ssem2, ssem3):
  c = lax.axis_index("c")
  s = lax.axis_index("s")

  gbufs = (g0, g1, g2, g3)
  rbufs = (row_v0, row_v1, row_v2, row_v3)
  vbufs = (val_v0, val_v1, val_v2, val_v3)
  gsems = (gsem0, gsem1, gsem2, gsem3)
  ssems = (ssem0, ssem1, ssem2, ssem3)

  pltpu.async_copy(col_hbm.at[pl.ds(s * E_PER_TILE, E_PER_TILE)],
                   col_v, gsem0)

  z = jnp.zeros((16,), jnp.float32)

  @plsc.parallel_loop(0, WCH, 1)
  def _zero(r):
    for q in range(DH // 16):
      g0[r, pl.ds(q * 16, 16)] = z

  for t in range(WCH_PER_TILE):
    m = s + t * NS

    @pl.when(m < N_WCH)
    def _():
      pltpu.sync_copy(g0.at[pl.ds(0, WCH)], acc_sh.at[pl.ds(m * WCH, WCH)])

  pltpu.make_async_copy(
      col_hbm.at[pl.ds(0, E_PER_TILE)], col_v, gsem0).wait()

  @plsc.parallel_loop(0, E_PER_TILE, 16, unroll=4)
  def _xform(k):
    col_v[pl.ds(k, 16)] = col_v[pl.ds(k, 16)] * 2 + c

  def fire_chunk(i, b):
    pltpu.async_copy(row_hbm.at[pl.ds(s * ROWS_PER_TILE + i * CH, CH)],
                     rbufs[b], gsems[b])
    pltpu.async_copy(val_hbm.at[pl.ds(s * E_PER_TILE + i * GROWS, GROWS)],
                     vbufs[b].at[pl.ds(0, GROWS)], gsems[b])
    pltpu.async_copy(x2_hbm.at[col_v.at[pl.ds(i * GROWS, GROWS)]],
                     gbufs[b].at[pl.ds(0, GROWS)], gsems[b])

  def drain_chunk(b):
    pltpu.make_async_copy(
        row_hbm.at[pl.ds(0, CH)], rbufs[b], gsems[b]).wait()
    pltpu.make_async_copy(
        val_hbm.at[pl.ds(0, GROWS)], vbufs[b].at[pl.ds(0, GROWS)],
        gsems[b]).wait()
    pltpu.make_async_copy(
        x2_hbm.at[col_v.at[pl.ds(0, GROWS)]],
        gbufs[b].at[pl.ds(0, GROWS)], gsems[b]).wait()

  def fire_scatters(b):
    for j in range(CH):
      pltpu.async_copy(gbufs[b].at[pl.ds(j * EW, EW)],
                       acc_sh.at[rbufs[b].at[j]], ssems[b], add=True)

  def drain_scatters(b):
    for j in range(CH):
      pltpu.make_async_copy(gbufs[b].at[pl.ds(j * EW, EW)],
                            acc_sh.at[rbufs[b].at[j]], ssems[b]).wait()

  def scale(b):
    g = gbufs[b]
    vb = vbufs[b]

    @plsc.parallel_loop(0, GROWS, 16, unroll=4)
    def _scale(r16):
      vv = vb[pl.ds(r16, 16)]
      for l in range(16):
        v = vv[l]
        for q in range(DH // 16):
          g[r16 + l, pl.ds(q * 16, 16)] = g[r16 + l, pl.ds(q * 16, 16)] * v

  def phase(i, k):
    @pl.when(i >= 2)
    def _():
      drain_scatters((k + 2) % NB)

    @pl.when(i + 2 < N_CHUNKS)
    def _():
      fire_chunk(i + 2, (k + 2) % NB)

    drain_chunk(k)
    scale(k)
    fire_scatters(k)

  fire_chunk(0, 0)
  fire_chunk(1, 1)
  plsc.subcore_barrier()

  @pl.loop(0, N_CHUNKS, step=NB)
  def _quad(i):
    for k in range(NB):
      phase(i + k, k)

  drain_scatters((N_CHUNKS - 2) % NB)
  drain_scatters((N_CHUNKS - 1) % NB)
  plsc.subcore_barrier()

  for t in range(WCH_PER_TILE):
    m = s + t * NS

    @pl.when(m < N_WCH)
    def _():
      r = m * WCH
      pltpu.sync_copy(acc_sh.at[pl.ds(r, WCH)],
                      out_hbm.at[pl.ds(r, WCH), pl.ds(c * DH, DH)])


@jax.jit
def _run(x2, row2, col1, val1):
  mesh = plsc.VectorSubcoreMesh(core_axis_name="c", subcore_axis_name="s")
  f = pl.kernel(
      _sc_body,
      out_type=jax.ShapeDtypeStruct((N, D), jnp.float32),
      mesh=mesh,
      scratch_types=(
          [pltpu.VMEM((E_PER_TILE,), jnp.int32)]
          + [pltpu.VMEM((CH, EW), jnp.int32)] * NB
          + [pltpu.VMEM((GPAD,), jnp.float32)] * NB
          + [pltpu.VMEM((GPAD, DH), jnp.float32)] * NB
          + [pltpu.VMEM_SHARED((N, DH), jnp.float32)]
          + [pltpu.SemaphoreType.DMA] * (2 * NB)
      ),
      compiler_params=pltpu.CompilerParams(use_tc_tiling_on_sc=False),
  )
  return f(x2, row2, col1, val1)


def kernel(indices, values, x):
  row = indices[0].astype(jnp.int32).reshape(ROWS, EW)
  col = indices[1].astype(jnp.int32).reshape(E)
  val = values.astype(jnp.float32).reshape(E)
  x2 = x.reshape(N * NC, DH)
  return _run(x2, row, col, val)


# speedup vs baseline: 13.7223x; 1.0077x over previous
import functools

import jax
import jax.numpy as jnp
from jax import lax
from jax.experimental import pallas as pl
from jax.experimental.pallas import tpu as pltpu
from jax.experimental.pallas import tpu_sc as plsc

N = 10000
E = 320000
D = 128

NC = 2
NS = 16
DH = D // NC

EW = 100
CH = 2
ROWS = E // EW
ROWS_PER_TILE = ROWS // NS
E_PER_TILE = ROWS_PER_TILE * EW
N_CHUNKS = ROWS_PER_TILE // CH
GROWS = CH * EW
GPAD = 208
NB = 4

WCH = 200
N_WCH = N // WCH
WCH_PER_TILE = -(-N_WCH // NS)


def _sc_body(x2_hbm, row_hbm, col_hbm, val_hbm, out_hbm,
             col_v, row_v0, row_v1, row_v2, row_v3,
             val_v0, val_v1, val_v2, val_v3, g0, g1, g2, g3, acc_sh,
             gsem0, gsem1, gsem2, gsem3, ssem0, ssem1, ssem2, ssem3):
  c = lax.axis_index("c")
  s = lax.axis_index("s")

  gbufs = (g0, g1, g2, g3)
  rbufs = (row_v0, row_v1, row_v2, row_v3)
  vbufs = (val_v0, val_v1, val_v2, val_v3)
  gsems = (gsem0, gsem1, gsem2, gsem3)
  ssems = (ssem0, ssem1, ssem2, ssem3)

  pltpu.async_copy(col_hbm.at[pl.ds(s * E_PER_TILE, E_PER_TILE)],
                   col_v, gsem0)

  z = jnp.zeros((16,), jnp.float32)

  @plsc.parallel_loop(0, WCH, 1)
  def _zero(r):
    for q in range(DH // 16):
      g0[r, pl.ds(q * 16, 16)] = z

  for t in range(WCH_PER_TILE):
    m = s + t * NS

    @pl.when(m < N_WCH)
    def _():
      pltpu.sync_copy(g0.at[pl.ds(0, WCH)], acc_sh.at[pl.ds(m * WCH, WCH)])

  pltpu.make_async_copy(
      col_hbm.at[pl.ds(0, E_PER_TILE)], col_v, gsem0).wait()


  def fire_chunk(i, b):
    pltpu.async_copy(row_hbm.at[pl.ds(s * ROWS_PER_TILE + i * CH, CH)],
                     rbufs[b], gsems[b])
    pltpu.async_copy(val_hbm.at[pl.ds(s * E_PER_TILE + i * GROWS, GROWS)],
                     vbufs[b].at[pl.ds(0, GROWS)], gsems[b])
    pltpu.async_copy(
        x2_hbm.at[pl.ds(c, NC * N - 1)].at[col_v.at[pl.ds(i * GROWS, GROWS)]],
        gbufs[b].at[pl.ds(0, GROWS)], gsems[b])

  def drain_chunk(b):
    pltpu.make_async_copy(
        row_hbm.at[pl.ds(0, CH)], rbufs[b], gsems[b]).wait()
    pltpu.make_async_copy(
        val_hbm.at[pl.ds(0, GROWS)], vbufs[b].at[pl.ds(0, GROWS)],
        gsems[b]).wait()
    pltpu.make_async_copy(
        x2_hbm.at[pl.ds(c, NC * N - 1)].at[col_v.at[pl.ds(0, GROWS)]],
        gbufs[b].at[pl.ds(0, GROWS)], gsems[b]).wait()

  def fire_scatters(b):
    for j in range(CH):
      pltpu.async_copy(gbufs[b].at[pl.ds(j * EW, EW)],
                       acc_sh.at[rbufs[b].at[j]], ssems[b], add=True)

  def drain_scatters(b):
    for j in range(CH):
      pltpu.make_async_copy(gbufs[b].at[pl.ds(j * EW, EW)],
                            acc_sh.at[rbufs[b].at[j]], ssems[b]).wait()

  def scale(b):
    g = gbufs[b]
    vb = vbufs[b]

    @plsc.parallel_loop(0, GROWS, 16, unroll=4)
    def _scale(r16):
      vv = vb[pl.ds(r16, 16)]
      for l in range(16):
        v = vv[l]
        for q in range(DH // 16):
          g[r16 + l, pl.ds(q * 16, 16)] = g[r16 + l, pl.ds(q * 16, 16)] * v

  def phase(i, k):
    @pl.when(i >= 2)
    def _():
      drain_scatters((k + 2) % NB)

    @pl.when(i + 2 < N_CHUNKS)
    def _():
      fire_chunk(i + 2, (k + 2) % NB)

    drain_chunk(k)
    scale(k)
    fire_scatters(k)

  fire_chunk(0, 0)
  fire_chunk(1, 1)
  plsc.subcore_barrier()

  @pl.loop(0, N_CHUNKS, step=NB)
  def _quad(i):
    for k in range(NB):
      phase(i + k, k)

  drain_scatters((N_CHUNKS - 2) % NB)
  drain_scatters((N_CHUNKS - 1) % NB)
  plsc.subcore_barrier()

  for t in range(WCH_PER_TILE):
    m = s + t * NS

    @pl.when(m < N_WCH)
    def _():
      r = m * WCH
      pltpu.sync_copy(acc_sh.at[pl.ds(r, WCH)],
                      out_hbm.at[pl.ds(r, WCH), pl.ds(c * DH, DH)])


@jax.jit
def _run(x2, row2, col1, val1):
  mesh = plsc.VectorSubcoreMesh(core_axis_name="c", subcore_axis_name="s")
  f = pl.kernel(
      _sc_body,
      out_type=jax.ShapeDtypeStruct((N, D), jnp.float32),
      mesh=mesh,
      scratch_types=(
          [pltpu.VMEM((E_PER_TILE,), jnp.int32)]
          + [pltpu.VMEM((CH, EW), jnp.int32)] * NB
          + [pltpu.VMEM((GPAD,), jnp.float32)] * NB
          + [pltpu.VMEM((GPAD, DH), jnp.float32)] * NB
          + [pltpu.VMEM_SHARED((N, DH), jnp.float32)]
          + [pltpu.SemaphoreType.DMA] * (2 * NB)
      ),
      compiler_params=pltpu.CompilerParams(use_tc_tiling_on_sc=False),
  )
  return f(x2, row2, col1, val1)


def kernel(indices, values, x):
  row = indices[0].astype(jnp.int32).reshape(ROWS, EW)
  col = indices[1].astype(jnp.int32).reshape(E) * 2
  val = values.astype(jnp.float32).reshape(E)
  x2 = x.reshape(N * NC, DH)
  return _run(x2, row, col, val)
